# baseline (speedup 1.0000x reference)
import functools

import jax
import jax.numpy as jnp
from jax import lax
from jax.experimental import pallas as pl
from jax.experimental.pallas import tpu as pltpu
from jax.experimental.pallas import tpu_sc as plsc

VOCAB = 1000000
EMBED_DIM = 64
BATCH = 16384
L = 16


@functools.partial(jax.jit, static_argnames=("num_workers",))
def _sc_embed_dot(x0, x1, w, *, num_workers):
    bpw = BATCH // num_workers
    mesh = plsc.VectorSubcoreMesh(core_axis_name="c", subcore_axis_name="s")
    num_cores = mesh.num_cores

    @functools.partial(
        pl.kernel,
        out_type=jax.ShapeDtypeStruct((BATCH,), jnp.float32),
        mesh=mesh,
        scratch_types=[
            pltpu.VMEM((bpw,), jnp.int32),
            pltpu.VMEM((bpw,), jnp.int32),
            pltpu.VMEM((bpw, EMBED_DIM), jnp.float32),
            pltpu.VMEM((bpw, EMBED_DIM), jnp.float32),
            pltpu.VMEM((bpw,), jnp.float32),
            pltpu.SemaphoreType.DMA,
            pltpu.SemaphoreType.DMA,
        ],
        compiler_params=pltpu.CompilerParams(
            needs_layout_passes=False, use_tc_tiling_on_sc=False),
    )
    def k(x0_hbm, x1_hbm, w_hbm, out_hbm,
          idx0_v, idx1_v, e0_v, e1_v, out_v, sem0, sem1):
        wid = lax.axis_index("s") * num_cores + lax.axis_index("c")
        base = wid * bpw
        pltpu.sync_copy(x0_hbm.at[pl.ds(base, bpw)], idx0_v)
        pltpu.sync_copy(x1_hbm.at[pl.ds(base, bpw)], idx1_v)
        c0 = pltpu.async_copy(w_hbm.at[idx0_v], e0_v, sem0)
        c1 = pltpu.async_copy(w_hbm.at[idx1_v], e1_v, sem1)
        c0.wait()
        c1.wait()

        def group(g, carry):
            rows = lax.iota(jnp.int32, L) + g * L
            acc = jnp.zeros((L,), jnp.float32)
            for d in range(EMBED_DIM):
                col = jnp.full((L,), d, jnp.int32)
                a = plsc.load_gather(e0_v, [rows, col])
                b = plsc.load_gather(e1_v, [rows, col])
                acc = acc + a * b
            out_v[pl.ds(g * L, L)] = 1.0 / (1.0 + jnp.exp(-acc))
            return carry

        lax.fori_loop(0, bpw // L, group, 0)
        pltpu.sync_copy(out_v, out_hbm.at[pl.ds(base, bpw)])

    return k(x0, x1, w)


def kernel(x, W_g):
    info = plsc.get_sparse_core_info()
    num_workers = info.num_cores * info.num_subcores
    out = _sc_embed_dot(x[:, 0], x[:, 1], W_g, num_workers=num_workers)
    return out.reshape(BATCH, 1)
